# Initial kernel scaffold; baseline (speedup 1.0000x reference)
import jax
import jax.numpy as jnp
from jax.experimental import pallas as pl


def kernel(h6_all, edge_index, edge_weight, W_self, W_nei, b1, Wp1, bp1, Wp2, bp2):
    raise NotImplementedError("write your pallas kernel here")


# speedup vs baseline: 4.0776x; 4.0776x over previous
import functools

import jax
import jax.numpy as jnp
from jax import lax
from jax.experimental import pallas as pl
from jax.experimental.pallas import tpu as pltpu
from jax.experimental.pallas import tpu_sc as plsc

N_NODES = 10000
N_EDGES = 160000
D = 256
DH = D // 2
NC = 2
NS = 16
LANES = 16
EPT = N_EDGES // NS
CHUNK = 80
NCHUNKS = EPT // CHUNK
ROW_BLOCKS = N_NODES // CHUNK


def _sc_call(h6r, src, dst, w):
    mesh = plsc.VectorSubcoreMesh(core_axis_name="c", subcore_axis_name="s")

    def body(h6r_hbm, src_hbm, dst_hbm, w_hbm, out_hbm, shared, srcbuf,
             dstbuf, wbuf, idxb, dstb, rows, sem):
        c = lax.axis_index("c")
        s = lax.axis_index("s")
        ebase = s * EPT

        pltpu.sync_copy(src_hbm.at[pl.ds(ebase, EPT)], srcbuf)
        pltpu.sync_copy(dst_hbm.at[pl.ds(ebase, EPT)], dstbuf)
        pltpu.sync_copy(w_hbm.at[pl.ds(ebase, EPT)], wbuf)

        def _zero_rows(r, _):
            for j in range(DH // LANES):
                rows[r, pl.ds(j * LANES, LANES)] = jnp.zeros((LANES,),
                                                             jnp.float32)
            return 0
        lax.fori_loop(0, CHUNK, _zero_rows, 0)

        nblk = (ROW_BLOCKS - s + NS - 1) // NS

        def _zero_blk(k, _):
            blk = s + k * NS
            pltpu.sync_copy(rows, shared.at[pl.ds(blk * CHUNK, CHUNK)])
            return 0
        lax.fori_loop(0, nblk, _zero_blk, 0)
        plsc.subcore_barrier()

        two = jnp.int32(2)

        def _chunk(ci, _):
            off = ci * CHUNK
            for j in range(CHUNK // LANES):
                sv = srcbuf[pl.ds(off + j * LANES, LANES)]
                idxb[pl.ds(j * LANES, LANES)] = sv * two + c
                dstb[pl.ds(j * LANES, LANES)] = dstbuf[
                    pl.ds(off + j * LANES, LANES)]
            pltpu.async_copy(h6r_hbm.at[idxb], rows, sem).wait()

            def _scale(g, _):
                wv = wbuf[pl.ds(off + g * LANES, LANES)]
                base = g * LANES
                for l in range(LANES):
                    wr = wv[l]
                    r = base + l
                    for j in range(DH // LANES):
                        sl = pl.ds(j * LANES, LANES)
                        rows[r, sl] = rows[r, sl] * wr
                return 0
            lax.fori_loop(0, CHUNK // LANES, _scale, 0)

            pltpu.sync_copy(rows, shared.at[dstb], add=True)
            return 0

        lax.fori_loop(0, NCHUNKS, _chunk, 0)
        plsc.subcore_barrier()

        def _out_blk(k, _):
            blk = s + k * NS
            r0 = blk * CHUNK
            pltpu.sync_copy(shared.at[pl.ds(r0, CHUNK)],
                            out_hbm.at[pl.ds(r0, CHUNK), c])
            return 0
        lax.fori_loop(0, nblk, _out_blk, 0)

    run = pl.kernel(
        body,
        out_type=jax.ShapeDtypeStruct((N_NODES, NC, DH), jnp.float32),
        mesh=mesh,
        scratch_types=[
            pltpu.VMEM_SHARED((N_NODES, DH), jnp.float32),
            pltpu.VMEM((EPT,), jnp.int32),
            pltpu.VMEM((EPT,), jnp.int32),
            pltpu.VMEM((EPT,), jnp.float32),
            pltpu.VMEM((CHUNK,), jnp.int32),
            pltpu.VMEM((CHUNK,), jnp.int32),
            pltpu.VMEM((CHUNK, DH), jnp.float32),
            pltpu.SemaphoreType.DMA,
        ],
    )
    return run(h6r, src, dst, w)


ROWS_BLK = 1000


def _tc_mlp_kernel(h6_ref, agg_ref, ws_ref, wn_ref, b1_ref, wp1_ref, bp1_ref,
                   wp2_ref, bp2_ref, out_ref):
    x = h6_ref[...]
    a = agg_ref[...]
    pre = (jnp.dot(a, wn_ref[...], preferred_element_type=jnp.float32)
           + jnp.dot(x, ws_ref[...], preferred_element_type=jnp.float32)
           + b1_ref[...])
    h7 = jnp.maximum(pre, 0.0)
    h = jnp.maximum(jnp.dot(h7, wp1_ref[...],
                            preferred_element_type=jnp.float32)
                    + bp1_ref[...], 0.0)
    out_ref[...] = (jnp.dot(h, wp2_ref[...],
                            preferred_element_type=jnp.float32)
                    + bp2_ref[...])


def _tc_mlp(h6_all, agg, W_self, W_nei, b1, Wp1, bp1, Wp2, bp2):
    grid = (N_NODES // ROWS_BLK,)
    row_spec = pl.BlockSpec((ROWS_BLK, D), lambda i: (i, 0))
    w_spec = pl.BlockSpec((D, D), lambda i: (0, 0))
    b_spec = pl.BlockSpec((1, D), lambda i: (0, 0))
    return pl.pallas_call(
        _tc_mlp_kernel,
        grid=grid,
        in_specs=[row_spec, row_spec, w_spec, w_spec, b_spec, w_spec, b_spec,
                  w_spec, b_spec],
        out_specs=row_spec,
        out_shape=jax.ShapeDtypeStruct((N_NODES, D), jnp.float32),
    )(h6_all, agg, W_self, W_nei, b1.reshape(1, D), Wp1, bp1.reshape(1, D),
      Wp2, bp2.reshape(1, D))


def kernel(h6_all, edge_index, edge_weight, W_self, W_nei, b1, Wp1, bp1, Wp2,
           bp2):
    src = edge_index[0].astype(jnp.int32)
    dst = edge_index[1].astype(jnp.int32)
    h6r = h6_all.reshape(2 * N_NODES, DH)
    agg = _sc_call(h6r, src, dst, edge_weight).reshape(N_NODES, D)
    return _tc_mlp(h6_all, agg, W_self, W_nei, b1, Wp1, bp1, Wp2, bp2)


# speedup vs baseline: 5.5306x; 1.3563x over previous
import functools

import jax
import jax.numpy as jnp
from jax import lax
from jax.experimental import pallas as pl
from jax.experimental.pallas import tpu as pltpu
from jax.experimental.pallas import tpu_sc as plsc

N_NODES = 10000
N_EDGES = 160000
D = 256
DH = D // 2
NC = 2
NS = 16
LANES = 16
EPT = N_EDGES // NS
CHUNK = 80
NCHUNKS = EPT // CHUNK
ROW_BLOCKS = N_NODES // CHUNK


def _sc_call(h6r, ei, w):
    mesh = plsc.VectorSubcoreMesh(core_axis_name="c", subcore_axis_name="s")

    def body(h6r_hbm, ei_hbm, w_hbm, out_hbm, shared,
             idxb0, idxb1, dstb0, dstb1, wt0, wt1, rows0, rows1,
             gsem0, gsem1, esem0, esem1):
        c = lax.axis_index("c")
        s = lax.axis_index("s")
        ebase = s * EPT
        idxbs, dstbs, wts = (idxb0, idxb1), (dstb0, dstb1), (wt0, wt1)
        rowss, gsems, esems = (rows0, rows1), (gsem0, gsem1), (esem0, esem1)

        def _zero_rows(r, _):
            for j in range(DH // LANES):
                rows0[r, pl.ds(j * LANES, LANES)] = jnp.zeros((LANES,),
                                                              jnp.float32)
            return 0
        lax.fori_loop(0, CHUNK, _zero_rows, 0)

        nblk = (ROW_BLOCKS - s + NS - 1) // NS

        def _zero_blk(k, _):
            blk = s + k * NS
            pltpu.sync_copy(rows0, shared.at[pl.ds(blk * CHUNK, CHUNK)])
            return 0
        lax.fori_loop(0, nblk, _zero_blk, 0)
        plsc.subcore_barrier()

        two = jnp.int32(2)

        def _eload(ci, b):
            off = ebase + ci * CHUNK
            pltpu.async_copy(ei_hbm.at[pl.ds(off, CHUNK)], idxbs[b],
                             esems[b])
            pltpu.async_copy(ei_hbm.at[pl.ds(N_EDGES + off, CHUNK)],
                             dstbs[b], esems[b])
            pltpu.async_copy(w_hbm.at[pl.ds(off, CHUNK)], wts[b], esems[b])

        def _ewait(ci, b):
            off = ebase + ci * CHUNK
            pltpu.make_async_copy(ei_hbm.at[pl.ds(off, CHUNK)], idxbs[b],
                                  esems[b]).wait()
            pltpu.make_async_copy(ei_hbm.at[pl.ds(N_EDGES + off, CHUNK)],
                                  dstbs[b], esems[b]).wait()
            pltpu.make_async_copy(w_hbm.at[pl.ds(off, CHUNK)], wts[b],
                                  esems[b]).wait()

        def _gather_start(b):
            idxb = idxbs[b]
            for j in range(CHUNK // LANES):
                sl = pl.ds(j * LANES, LANES)
                idxb[sl] = idxb[sl] * two + c
            pltpu.async_copy(h6r_hbm.at[idxb], rowss[b], gsems[b])

        def _scale(rows, wt):
            def _grp(g, _):
                wv = wt[pl.ds(g * LANES, LANES)]
                base = g * LANES
                for l in range(LANES):
                    wr = wv[l]
                    r = base + l
                    for j in range(DH // LANES):
                        sl = pl.ds(j * LANES, LANES)
                        rows[r, sl] = rows[r, sl] * wr
                return 0
            lax.fori_loop(0, CHUNK // LANES, _grp, 0)

        def _step(k, b):
            bb = 1 - b
            pltpu.make_async_copy(h6r_hbm.at[idxbs[b]], rowss[b],
                                  gsems[b]).wait()

            @pl.when(k + 1 < NCHUNKS)
            def _():
                _ewait(k + 1, bb)
                _gather_start(bb)

            _scale(rowss[b], wts[b])
            pltpu.sync_copy(rowss[b], shared.at[dstbs[b]], add=True)

            @pl.when(k + 2 < NCHUNKS)
            def _():
                _eload(k + 2, b)

        _eload(0, 0)
        _eload(1, 1)
        _ewait(0, 0)
        _gather_start(0)

        def _pair(ii, _):
            _step(ii * 2, 0)
            _step(ii * 2 + 1, 1)
            return 0

        lax.fori_loop(0, NCHUNKS // 2, _pair, 0)
        _step(NCHUNKS - 1, 0)

        plsc.subcore_barrier()

        def _out_blk(k, _):
            blk = s + k * NS
            r0 = blk * CHUNK
            pltpu.sync_copy(shared.at[pl.ds(r0, CHUNK)],
                            out_hbm.at[pl.ds(r0, CHUNK), c])
            return 0
        lax.fori_loop(0, nblk, _out_blk, 0)

    run = pl.kernel(
        body,
        out_type=jax.ShapeDtypeStruct((N_NODES, NC, DH), jnp.float32),
        mesh=mesh,
        scratch_types=[
            pltpu.VMEM_SHARED((N_NODES, DH), jnp.float32),
            pltpu.VMEM((CHUNK,), jnp.int32),
            pltpu.VMEM((CHUNK,), jnp.int32),
            pltpu.VMEM((CHUNK,), jnp.int32),
            pltpu.VMEM((CHUNK,), jnp.int32),
            pltpu.VMEM((CHUNK,), jnp.float32),
            pltpu.VMEM((CHUNK,), jnp.float32),
            pltpu.VMEM((CHUNK, DH), jnp.float32),
            pltpu.VMEM((CHUNK, DH), jnp.float32),
            pltpu.SemaphoreType.DMA,
            pltpu.SemaphoreType.DMA,
            pltpu.SemaphoreType.DMA,
            pltpu.SemaphoreType.DMA,
        ],
    )
    return run(h6r, ei, w)


ROWS_BLK = 1000


def _tc_mlp_kernel(h6_ref, agg_ref, ws_ref, wn_ref, b1_ref, wp1_ref, bp1_ref,
                   wp2_ref, bp2_ref, out_ref):
    x = h6_ref[...]
    a = agg_ref[...]
    pre = (jnp.dot(a, wn_ref[...], preferred_element_type=jnp.float32)
           + jnp.dot(x, ws_ref[...], preferred_element_type=jnp.float32)
           + b1_ref[...])
    h7 = jnp.maximum(pre, 0.0)
    h = jnp.maximum(jnp.dot(h7, wp1_ref[...],
                            preferred_element_type=jnp.float32)
                    + bp1_ref[...], 0.0)
    out_ref[...] = (jnp.dot(h, wp2_ref[...],
                            preferred_element_type=jnp.float32)
                    + bp2_ref[...])


def _tc_mlp(h6_all, agg, W_self, W_nei, b1, Wp1, bp1, Wp2, bp2):
    grid = (N_NODES // ROWS_BLK,)
    row_spec = pl.BlockSpec((ROWS_BLK, D), lambda i: (i, 0))
    w_spec = pl.BlockSpec((D, D), lambda i: (0, 0))
    b_spec = pl.BlockSpec((1, D), lambda i: (0, 0))
    return pl.pallas_call(
        _tc_mlp_kernel,
        grid=grid,
        in_specs=[row_spec, row_spec, w_spec, w_spec, b_spec, w_spec, b_spec,
                  w_spec, b_spec],
        out_specs=row_spec,
        out_shape=jax.ShapeDtypeStruct((N_NODES, D), jnp.float32),
    )(h6_all, agg, W_self, W_nei, b1.reshape(1, D), Wp1, bp1.reshape(1, D),
      Wp2, bp2.reshape(1, D))


def kernel(h6_all, edge_index, edge_weight, W_self, W_nei, b1, Wp1, bp1, Wp2,
           bp2):
    ei = edge_index.astype(jnp.int32).reshape(2 * N_EDGES)
    h6r = h6_all.reshape(2 * N_NODES, DH)
    agg = _sc_call(h6r, ei, edge_weight).reshape(N_NODES, D)
    return _tc_mlp(h6_all, agg, W_self, W_nei, b1, Wp1, bp1, Wp2, bp2)


# speedup vs baseline: 6.1037x; 1.1036x over previous
import functools

import jax
import jax.numpy as jnp
from jax import lax
from jax.experimental import pallas as pl
from jax.experimental.pallas import tpu as pltpu
from jax.experimental.pallas import tpu_sc as plsc

N_NODES = 10000
N_EDGES = 160000
D = 256
DH = D // 2
NC = 2
NS = 16
LANES = 16
EPT = N_EDGES // NS
CHUNK = 80
NCHUNKS = EPT // CHUNK
ROW_BLOCKS = N_NODES // CHUNK


def _sc_call(h6r, ei, w):
    mesh = plsc.VectorSubcoreMesh(core_axis_name="c", subcore_axis_name="s")

    def body(h6r_hbm, ei_hbm, w_hbm, out_hbm, shared,
             idxb0, idxb1, dstb0, dstb1, sdst0, sdst1, wt0, wt1,
             rows0, rows1, gsem0, gsem1, esem0, esem1, ssem0, ssem1):
        c = lax.axis_index("c")
        s = lax.axis_index("s")
        ebase = s * EPT
        idxbs, dstbs, wts = (idxb0, idxb1), (dstb0, dstb1), (wt0, wt1)
        sdsts = (sdst0, sdst1)
        rowss, gsems, esems = (rows0, rows1), (gsem0, gsem1), (esem0, esem1)
        ssems = (ssem0, ssem1)

        @plsc.parallel_loop(0, CHUNK)
        def _zero_rows(r):
            for j in range(DH // LANES):
                rows0[r, pl.ds(j * LANES, LANES)] = jnp.zeros((LANES,),
                                                              jnp.float32)

        nblk = (ROW_BLOCKS - s + NS - 1) // NS

        def _zero_blk(k, _):
            blk = s + k * NS
            pltpu.async_copy(rows0, shared.at[pl.ds(blk * CHUNK, CHUNK)],
                             gsem0)
            return 0
        lax.fori_loop(0, nblk, _zero_blk, 0)

        def _zero_drain(k, _):
            blk = s + k * NS
            pltpu.make_async_copy(rows0,
                                  shared.at[pl.ds(blk * CHUNK, CHUNK)],
                                  gsem0).wait()
            return 0
        lax.fori_loop(0, nblk, _zero_drain, 0)
        plsc.subcore_barrier()

        two = jnp.int32(2)

        def _eload(ci, b):
            off = ebase + ci * CHUNK
            pltpu.async_copy(ei_hbm.at[pl.ds(off, CHUNK)], idxbs[b],
                             esems[b])
            pltpu.async_copy(ei_hbm.at[pl.ds(N_EDGES + off, CHUNK)],
                             dstbs[b], esems[b])
            pltpu.async_copy(w_hbm.at[pl.ds(off, CHUNK)], wts[b], esems[b])

        def _ewait(ci, b):
            off = ebase + ci * CHUNK
            pltpu.make_async_copy(ei_hbm.at[pl.ds(off, CHUNK)], idxbs[b],
                                  esems[b]).wait()
            pltpu.make_async_copy(ei_hbm.at[pl.ds(N_EDGES + off, CHUNK)],
                                  dstbs[b], esems[b]).wait()
            pltpu.make_async_copy(w_hbm.at[pl.ds(off, CHUNK)], wts[b],
                                  esems[b]).wait()

        def _gather_start(b):
            idxb = idxbs[b]
            for j in range(CHUNK // LANES):
                sl = pl.ds(j * LANES, LANES)
                idxb[sl] = idxb[sl] * two + c
            pltpu.async_copy(h6r_hbm.at[idxb], rowss[b], gsems[b])

        def _scale(rows, wt):
            @plsc.parallel_loop(0, CHUNK // LANES)
            def _grp(g):
                wv = wt[pl.ds(g * LANES, LANES)]
                base = g * LANES
                for l in range(LANES):
                    wr = wv[l]
                    r = base + l
                    for j in range(DH // LANES):
                        sl = pl.ds(j * LANES, LANES)
                        rows[r, sl] = rows[r, sl] * wr

        def _scat_wait(b):
            pltpu.make_async_copy(rowss[b], shared.at[sdsts[b]],
                                  ssems[b]).wait()

        def _step(k, b):
            bb = 1 - b
            pltpu.make_async_copy(h6r_hbm.at[idxbs[b]], rowss[b],
                                  gsems[b]).wait()

            @pl.when(k + 1 < NCHUNKS)
            def _():
                @pl.when(k >= 1)
                def _():
                    _scat_wait(bb)
                _ewait(k + 1, bb)
                _gather_start(bb)

            _scale(rowss[b], wts[b])
            for j in range(CHUNK // LANES):
                sl = pl.ds(j * LANES, LANES)
                sdsts[b][sl] = dstbs[b][sl]
            pltpu.async_copy(rowss[b], shared.at[sdsts[b]], ssems[b],
                             add=True)

            @pl.when(k + 2 < NCHUNKS)
            def _():
                _eload(k + 2, b)

        _eload(0, 0)
        _eload(1, 1)
        _ewait(0, 0)
        _gather_start(0)

        def _pair(ii, _):
            _step(ii * 2, 0)
            _step(ii * 2 + 1, 1)
            return 0

        lax.fori_loop(0, NCHUNKS // 2, _pair, 0)
        _step(NCHUNKS - 1, 0)
        _scat_wait(1)
        _scat_wait(0)

        plsc.subcore_barrier()

        def _out_blk(k, _):
            blk = s + k * NS
            r0 = blk * CHUNK
            pltpu.async_copy(shared.at[pl.ds(r0, CHUNK)],
                             out_hbm.at[pl.ds(r0, CHUNK), c], gsem0)
            return 0
        lax.fori_loop(0, nblk, _out_blk, 0)

        def _out_drain(k, _):
            blk = s + k * NS
            r0 = blk * CHUNK
            pltpu.make_async_copy(shared.at[pl.ds(r0, CHUNK)],
                                  out_hbm.at[pl.ds(r0, CHUNK), c],
                                  gsem0).wait()
            return 0
        lax.fori_loop(0, nblk, _out_drain, 0)

    run = pl.kernel(
        body,
        out_type=jax.ShapeDtypeStruct((N_NODES, NC, DH), jnp.float32),
        mesh=mesh,
        scratch_types=[
            pltpu.VMEM_SHARED((N_NODES, DH), jnp.float32),
            pltpu.VMEM((CHUNK,), jnp.int32),
            pltpu.VMEM((CHUNK,), jnp.int32),
            pltpu.VMEM((CHUNK,), jnp.int32),
            pltpu.VMEM((CHUNK,), jnp.int32),
            pltpu.VMEM((CHUNK,), jnp.int32),
            pltpu.VMEM((CHUNK,), jnp.int32),
            pltpu.VMEM((CHUNK,), jnp.float32),
            pltpu.VMEM((CHUNK,), jnp.float32),
            pltpu.VMEM((CHUNK, DH), jnp.float32),
            pltpu.VMEM((CHUNK, DH), jnp.float32),
            pltpu.SemaphoreType.DMA,
            pltpu.SemaphoreType.DMA,
            pltpu.SemaphoreType.DMA,
            pltpu.SemaphoreType.DMA,
            pltpu.SemaphoreType.DMA,
            pltpu.SemaphoreType.DMA,
        ],
    )
    return run(h6r, ei, w)


ROWS_BLK = 1000


def _tc_mlp_kernel(h6_ref, agg_ref, ws_ref, wn_ref, b1_ref, wp1_ref, bp1_ref,
                   wp2_ref, bp2_ref, out_ref):
    x = h6_ref[...]
    a = agg_ref[...]
    pre = (jnp.dot(a, wn_ref[...], preferred_element_type=jnp.float32)
           + jnp.dot(x, ws_ref[...], preferred_element_type=jnp.float32)
           + b1_ref[...])
    h7 = jnp.maximum(pre, 0.0)
    h = jnp.maximum(jnp.dot(h7, wp1_ref[...],
                            preferred_element_type=jnp.float32)
                    + bp1_ref[...], 0.0)
    out_ref[...] = (jnp.dot(h, wp2_ref[...],
                            preferred_element_type=jnp.float32)
                    + bp2_ref[...])


def _tc_mlp(h6_all, agg, W_self, W_nei, b1, Wp1, bp1, Wp2, bp2):
    grid = (N_NODES // ROWS_BLK,)
    row_spec = pl.BlockSpec((ROWS_BLK, D), lambda i: (i, 0))
    w_spec = pl.BlockSpec((D, D), lambda i: (0, 0))
    b_spec = pl.BlockSpec((1, D), lambda i: (0, 0))
    return pl.pallas_call(
        _tc_mlp_kernel,
        grid=grid,
        in_specs=[row_spec, row_spec, w_spec, w_spec, b_spec, w_spec, b_spec,
                  w_spec, b_spec],
        out_specs=row_spec,
        out_shape=jax.ShapeDtypeStruct((N_NODES, D), jnp.float32),
    )(h6_all, agg, W_self, W_nei, b1.reshape(1, D), Wp1, bp1.reshape(1, D),
      Wp2, bp2.reshape(1, D))


def kernel(h6_all, edge_index, edge_weight, W_self, W_nei, b1, Wp1, bp1, Wp2,
           bp2):
    ei = edge_index.astype(jnp.int32).reshape(2 * N_EDGES)
    h6r = h6_all.reshape(2 * N_NODES, DH)
    agg = _sc_call(h6r, ei, edge_weight).reshape(N_NODES, D)
    return _tc_mlp(h6_all, agg, W_self, W_nei, b1, Wp1, bp1, Wp2, bp2)
